# baseline (speedup 1.0000x reference)
import functools

import jax
import jax.numpy as jnp
from jax import lax
from jax.experimental import pallas as pl
from jax.experimental.pallas import tpu as pltpu
from jax.experimental.pallas import tpu_sc as plsc

D = 128
BLOCK_E = 20000
CHUNK = 512
SUB = CHUNK // 128


def _matmul_body(y_ref, w_ref, out_ref):
    out_ref[...] = lax.dot_general(
        y_ref[...], w_ref[...],
        dimension_numbers=(((1,), (1,)), ((), ())),
        preferred_element_type=jnp.float32)


def _tc_matmul(Y, W):
    E = Y.shape[0]
    return pl.pallas_call(
        _matmul_body,
        grid=(E // BLOCK_E,),
        in_specs=[
            pl.BlockSpec((BLOCK_E, D), lambda i: (i, 0)),
            pl.BlockSpec((D, D), lambda i: (0, 0)),
        ],
        out_specs=pl.BlockSpec((BLOCK_E, D), lambda i: (i, 0)),
        out_shape=jax.ShapeDtypeStruct((E, D), jnp.float32),
    )(Y, W)


def _final_body(x_ref, agg_ref, w_ref, out_ref):
    out_ref[...] = x_ref[...] + lax.dot_general(
        agg_ref[...], w_ref[...],
        dimension_numbers=(((1,), (1,)), ((), ())),
        preferred_element_type=jnp.float32,
        precision=lax.Precision.HIGHEST)


def _tc_final(X, agg, W, block_n=2000):
    n_nodes = X.shape[0]
    return pl.pallas_call(
        _final_body,
        grid=(n_nodes // block_n,),
        in_specs=[
            pl.BlockSpec((block_n, D), lambda i: (i, 0)),
            pl.BlockSpec((block_n, D), lambda i: (i, 0)),
            pl.BlockSpec((D, D), lambda i: (0, 0)),
        ],
        out_specs=pl.BlockSpec((block_n, D), lambda i: (i, 0)),
        out_shape=jax.ShapeDtypeStruct((n_nodes, D), jnp.float32),
    )(X, agg, W)


def _sc_scatter(Yraw, src2d, n_nodes):
    E = Yraw.shape[0]
    info = plsc.get_sparse_core_info()
    nc, ns = info.num_cores, info.num_subcores
    half = D // nc
    rows_per_tile = n_nodes // ns
    n_chunks = E // CHUNK
    q, r = divmod(n_chunks, ns)

    mesh = plsc.VectorSubcoreMesh(core_axis_name="c", subcore_axis_name="s")

    @functools.partial(
        pl.kernel,
        mesh=mesh,
        compiler_params=pltpu.CompilerParams(use_tc_tiling_on_sc=False),
        out_type=jax.ShapeDtypeStruct((n_nodes, D), jnp.float32),
        scratch_types=[
            pltpu.VMEM_SHARED((n_nodes, half), jnp.float32),
            pltpu.VMEM((2, SUB, 128), jnp.int32),
            pltpu.VMEM((2, CHUNK, half), jnp.float32),
            pltpu.SemaphoreType.DMA((2,)),
            pltpu.SemaphoreType.DMA((2,)),
        ],
    )
    def scatter_kernel(yp_hbm, src_hbm, out_hbm, acc_sh, idx_v, rows_v,
                       sem_i, sem_r):
        c = lax.axis_index("c")
        s = lax.axis_index("s")
        r0 = s * rows_per_tile
        col0 = c * half

        zv = jnp.zeros((16,), jnp.float32)
        lanes = half // 16

        def zb(i, carry):
            rows_v[0, lax.div(i, lanes), pl.ds(lax.rem(i, lanes) * 16, 16)] = zv
            return carry

        lax.fori_loop(0, 128 * lanes, zb, jnp.int32(0))
        for p in range(4):
            pltpu.sync_copy(rows_v.at[0, pl.ds(0, 128)],
                            acc_sh.at[pl.ds(r0 + p * 128, 128)])
        pltpu.sync_copy(
            rows_v.at[0, pl.ds(0, rows_per_tile - 512)],
            acc_sh.at[pl.ds(r0 + 512, rows_per_tile - 512)])
        plsc.subcore_barrier()

        nk = jnp.where(s < r, q + 1, q)

        def fetch(k, b):
            ch = s + k * ns
            pltpu.async_copy(
                src_hbm.at[pl.ds(ch * SUB, SUB)], idx_v.at[b], sem_i.at[b])
            pltpu.async_copy(
                yp_hbm.at[pl.ds(ch * CHUNK, CHUNK), pl.ds(col0, half)],
                rows_v.at[b], sem_r.at[b])

        fetch(jnp.int32(0), jnp.int32(0))

        def body(k, carry):
            b = lax.rem(k, 2)

            @pl.when(k + 1 < nk)
            def _():
                fetch(k + 1, lax.rem(k + 1, 2))

            ch = s + k * ns
            pltpu.make_async_copy(
                src_hbm.at[pl.ds(ch * SUB, SUB)], idx_v.at[b],
                sem_i.at[b]).wait()
            pltpu.make_async_copy(
                yp_hbm.at[pl.ds(ch * CHUNK, CHUNK), pl.ds(col0, half)],
                rows_v.at[b], sem_r.at[b]).wait()
            for j in range(SUB):
                pltpu.sync_copy(
                    rows_v.at[b, pl.ds(j * 128, 128)],
                    acc_sh.at[idx_v.at[b, j]],
                    add=True)
            return carry

        lax.fori_loop(0, nk, body, jnp.int32(0))
        plsc.subcore_barrier()

        pltpu.sync_copy(
            acc_sh.at[pl.ds(r0, rows_per_tile)],
            out_hbm.at[pl.ds(r0, rows_per_tile), pl.ds(col0, half)])

    return scatter_kernel(Yraw, src2d)


def kernel(X, Y, edge_index, W):
    n_nodes = X.shape[0]
    src2d = edge_index[:, 0].reshape(-1, 128)
    agg = _sc_scatter(Y, src2d, n_nodes)
    Yp = _tc_matmul(Y, W)
    Xp = _tc_final(X, agg, W)
    return (Xp, Yp)


# speedup vs baseline: 1.0001x; 1.0001x over previous
import functools

import jax
import jax.numpy as jnp
from jax import lax
from jax.experimental import pallas as pl
from jax.experimental.pallas import tpu as pltpu
from jax.experimental.pallas import tpu_sc as plsc

D = 128
BLOCK_E = 20000
CHUNK = 512
SUB = CHUNK // 128


def _matmul_body(y_ref, w_ref, out_ref):
    out_ref[...] = lax.dot_general(
        y_ref[...], w_ref[...],
        dimension_numbers=(((1,), (1,)), ((), ())),
        preferred_element_type=jnp.float32)


def _tc_matmul(Y, W):
    E = Y.shape[0]
    return pl.pallas_call(
        _matmul_body,
        grid=(E // BLOCK_E,),
        in_specs=[
            pl.BlockSpec((BLOCK_E, D), lambda i: (i, 0)),
            pl.BlockSpec((D, D), lambda i: (0, 0)),
        ],
        out_specs=pl.BlockSpec((BLOCK_E, D), lambda i: (i, 0)),
        out_shape=jax.ShapeDtypeStruct((E, D), jnp.float32),
    )(Y, W)


def _final_body(x_ref, agg_ref, w_ref, out_ref):
    out_ref[...] = x_ref[...] + lax.dot_general(
        agg_ref[...], w_ref[...],
        dimension_numbers=(((1,), (1,)), ((), ())),
        preferred_element_type=jnp.float32,
        precision=lax.Precision.HIGHEST)


def _tc_final(X, agg, W, block_n=2000):
    n_nodes = X.shape[0]
    return pl.pallas_call(
        _final_body,
        grid=(n_nodes // block_n,),
        in_specs=[
            pl.BlockSpec((block_n, D), lambda i: (i, 0)),
            pl.BlockSpec((block_n, D), lambda i: (i, 0)),
            pl.BlockSpec((D, D), lambda i: (0, 0)),
        ],
        out_specs=pl.BlockSpec((block_n, D), lambda i: (i, 0)),
        out_shape=jax.ShapeDtypeStruct((n_nodes, D), jnp.float32),
    )(X, agg, W)


def _sc_scatter(Yraw, src2d, n_nodes):
    E = Yraw.shape[0]
    info = plsc.get_sparse_core_info()
    nc, ns = info.num_cores, info.num_subcores
    half = D // nc
    rows_per_tile = n_nodes // ns
    n_chunks = E // CHUNK
    q, r = divmod(n_chunks, ns)

    mesh = plsc.VectorSubcoreMesh(core_axis_name="c", subcore_axis_name="s")

    @functools.partial(
        pl.kernel,
        mesh=mesh,
        compiler_params=pltpu.CompilerParams(use_tc_tiling_on_sc=False),
        out_type=jax.ShapeDtypeStruct((n_nodes, D), jnp.float32),
        scratch_types=[
            pltpu.VMEM_SHARED((n_nodes, half), jnp.float32),
            pltpu.VMEM((SUB, 128), jnp.int32),
            pltpu.VMEM((SUB, 128), jnp.int32),
            pltpu.VMEM((CHUNK, half), jnp.float32),
            pltpu.VMEM((CHUNK, half), jnp.float32),
            pltpu.SemaphoreType.DMA,
            pltpu.SemaphoreType.DMA,
            pltpu.SemaphoreType.DMA,
            pltpu.SemaphoreType.DMA,
        ],
    )
    def scatter_kernel(yp_hbm, src_hbm, out_hbm, acc_sh, idx_v0, idx_v1,
                       rows_v0, rows_v1, sem_i0, sem_i1, sem_r0, sem_r1):
        c = lax.axis_index("c")
        s = lax.axis_index("s")
        r0 = s * rows_per_tile
        col0 = c * half
        bufs = ((idx_v0, rows_v0, sem_i0, sem_r0),
                (idx_v1, rows_v1, sem_i1, sem_r1))

        zv = jnp.zeros((16,), jnp.float32)
        lanes = half // 16

        def zb(i, carry):
            rows_v0[lax.div(i, lanes), pl.ds(lax.rem(i, lanes) * 16, 16)] = zv
            return carry

        lax.fori_loop(0, 128 * lanes, zb, jnp.int32(0))
        for p in range(4):
            pltpu.sync_copy(rows_v0.at[pl.ds(0, 128)],
                            acc_sh.at[pl.ds(r0 + p * 128, 128)])
        pltpu.sync_copy(
            rows_v0.at[pl.ds(0, rows_per_tile - 512)],
            acc_sh.at[pl.ds(r0 + 512, rows_per_tile - 512)])
        plsc.subcore_barrier()

        nk = jnp.where(s < r, q + 1, q)

        def fetch(k, buf):
            idxb, rowb, semi, semr = buf
            ch = s + k * ns
            pltpu.async_copy(src_hbm.at[pl.ds(ch * SUB, SUB)], idxb, semi)
            pltpu.async_copy(
                yp_hbm.at[pl.ds(ch * CHUNK, CHUNK), pl.ds(col0, half)],
                rowb, semr)

        def consume(k, buf):
            idxb, rowb, semi, semr = buf
            ch = s + k * ns
            pltpu.make_async_copy(
                src_hbm.at[pl.ds(ch * SUB, SUB)], idxb, semi).wait()
            pltpu.make_async_copy(
                yp_hbm.at[pl.ds(ch * CHUNK, CHUNK), pl.ds(col0, half)],
                rowb, semr).wait()
            for j in range(SUB):
                pltpu.sync_copy(
                    rowb.at[pl.ds(j * 128, 128)],
                    acc_sh.at[idxb.at[j]],
                    add=True)

        fetch(jnp.int32(0), bufs[0])

        def body(p, carry):
            for b in (0, 1):
                k = 2 * p + b

                @pl.when(k < nk)
                def _(k=k, b=b):
                    @pl.when(k + 1 < nk)
                    def _():
                        fetch(k + 1, bufs[1 - b])

                    consume(k, bufs[b])
            return carry

        n_pairs = (q + 1 + 1) // 2
        lax.fori_loop(0, n_pairs, body, jnp.int32(0))
        plsc.subcore_barrier()

        pltpu.sync_copy(
            acc_sh.at[pl.ds(r0, rows_per_tile)],
            out_hbm.at[pl.ds(r0, rows_per_tile), pl.ds(col0, half)])

    return scatter_kernel(Yraw, src2d)


def kernel(X, Y, edge_index, W):
    n_nodes = X.shape[0]
    src2d = edge_index[:, 0].reshape(-1, 128)
    agg = _sc_scatter(Y, src2d, n_nodes)
    Yp = _tc_matmul(Y, W)
    Xp = _tc_final(X, agg, W)
    return (Xp, Yp)
